# Initial kernel scaffold; baseline (speedup 1.0000x reference)
import jax
import jax.numpy as jnp
from jax.experimental import pallas as pl


def kernel(hidden_states, base_perm, shifts):
    raise NotImplementedError("write your pallas kernel here")


# speedup vs baseline: 1.8107x; 1.8107x over previous
import functools

import jax
import jax.numpy as jnp
from jax import lax
from jax.experimental import pallas as pl
from jax.experimental.pallas import tpu as pltpu
from jax.experimental.pallas import tpu_sc as plsc

NC, NS, LANES = 2, 16, 16
NW = NC * NS


def _body(B, L, D, rows_per_w, CH, hs_hbm, perm_hbm, shifts_hbm, out_hbm,
          perm_v, shifts_v, gidx_v, bufs, idx_sem, sems):
    wid = lax.axis_index("s") * NC + lax.axis_index("c")
    base_row = wid * rows_per_w
    b = base_row // L
    r0 = base_row % L

    pltpu.async_copy(perm_hbm, perm_v, idx_sem).wait()
    pltpu.async_copy(shifts_hbm, shifts_v, idx_sem).wait()
    shift = plsc.load_gather(shifts_v, [jnp.full((LANES,), b, jnp.int32)])
    row_base = b * L

    def idx_body(j, _):
        pos = (lax.iota(jnp.int32, LANES) + shift + (r0 + j * LANES)) & (L - 1)
        vals = plsc.load_gather(perm_v, [pos])
        gidx_v[pl.ds(j * LANES, LANES)] = vals + row_base
        return ()

    lax.fori_loop(0, rows_per_w // LANES, idx_body, (), unroll=4)

    nchunks = rows_per_w // CH
    nbuf = len(bufs)

    def fire(c, slot):
        pltpu.async_copy(
            hs_hbm.at[gidx_v.at[pl.ds(c * CH, CH)]], bufs[slot], sems[slot])

    def wait_gather(slot):
        pltpu.make_async_copy(hs_hbm.at[pl.ds(0, CH)], bufs[slot],
                              sems[slot]).wait()

    for s in range(nbuf):
        fire(s, s)

    def ring_body(i, _):
        for s in range(nbuf):
            c = i * nbuf + s
            wait_gather(s)
            pltpu.sync_copy(bufs[s], out_hbm.at[pl.ds(base_row + c * CH, CH)])

            @pl.when(c + nbuf < nchunks)
            def _():
                fire(c + nbuf, s)
        return ()

    lax.fori_loop(0, nchunks // nbuf, ring_body, ())


def kernel(hidden_states, base_perm, shifts):
    B, L, D = hidden_states.shape
    assert (B * L) % NW == 0 and L & (L - 1) == 0
    rows_per_w = (B * L) // NW
    CH = 32
    NBUF = 2

    hs2 = hidden_states.reshape(B * L, D)
    perm = base_perm.astype(jnp.int32)
    shifts16 = jnp.zeros((LANES,), jnp.int32).at[:B].set(shifts.astype(jnp.int32))

    mesh = plsc.VectorSubcoreMesh(core_axis_name="c", subcore_axis_name="s")
    run = pl.kernel(
        functools.partial(_body, B, L, D, rows_per_w, CH),
        out_type=jax.ShapeDtypeStruct((B * L, D), jnp.float32),
        mesh=mesh,
        scratch_types=[
            pltpu.VMEM((L,), jnp.int32),
            pltpu.VMEM((LANES,), jnp.int32),
            pltpu.VMEM((rows_per_w,), jnp.int32),
            [pltpu.VMEM((CH, D), jnp.float32) for _ in range(NBUF)],
            pltpu.SemaphoreType.DMA,
            [pltpu.SemaphoreType.DMA for _ in range(NBUF)],
        ],
        compiler_params=pltpu.CompilerParams(needs_layout_passes=False),
    )
    out = run(hs2, perm, shifts16)
    return out.reshape(B, L, D)


# speedup vs baseline: 1.8116x; 1.0005x over previous
import functools

import jax
import jax.numpy as jnp
from jax import lax
from jax.experimental import pallas as pl
from jax.experimental.pallas import tpu as pltpu
from jax.experimental.pallas import tpu_sc as plsc

NC, NS, LANES = 2, 16, 16
NW = NC * NS


def _body(B, L, D, rows_per_w, CH, hs_hbm, perm_hbm, shifts_hbm, out_hbm,
          perm_v, shifts_v, gidx_v, bufs, idx_sem, sems):
    wid = lax.axis_index("s") * NC + lax.axis_index("c")
    base_row = wid * rows_per_w
    b = base_row // L
    r0 = base_row % L

    pltpu.async_copy(perm_hbm, perm_v, idx_sem).wait()
    pltpu.async_copy(shifts_hbm, shifts_v, idx_sem).wait()
    shift = plsc.load_gather(shifts_v, [jnp.full((LANES,), b, jnp.int32)])
    row_base = b * L

    def idx_body(j, _):
        pos = (lax.iota(jnp.int32, LANES) + shift + (r0 + j * LANES)) & (L - 1)
        vals = plsc.load_gather(perm_v, [pos])
        gidx_v[pl.ds(j * LANES, LANES)] = vals + row_base
        return ()

    lax.fori_loop(0, rows_per_w // LANES, idx_body, (), unroll=4)

    nchunks = rows_per_w // CH
    nbuf = len(bufs)
    gsems, wsems = sems

    def fire_gather(c, slot):
        pltpu.async_copy(
            hs_hbm.at[gidx_v.at[pl.ds(c * CH, CH)]], bufs[slot], gsems[slot])

    def wait_gather(slot):
        pltpu.make_async_copy(hs_hbm.at[pl.ds(0, CH)], bufs[slot],
                              gsems[slot]).wait()

    def fire_write(c, slot):
        pltpu.async_copy(
            bufs[slot], out_hbm.at[pl.ds(base_row + c * CH, CH)], wsems[slot])

    def wait_write(slot):
        pltpu.make_async_copy(bufs[slot], out_hbm.at[pl.ds(base_row, CH)],
                              wsems[slot]).wait()

    for s in range(min(nbuf, nchunks)):
        fire_gather(s, s)

    ncycles = -(-nchunks // nbuf)

    def ring_body(i, _):
        for s in range(nbuf):
            c = i * nbuf + s

            @pl.when(c < nchunks)
            def _():
                wait_gather(s)
                fire_write(c, s)

            @pl.when(c + nbuf < nchunks)
            def _():
                wait_write(s)
                fire_gather(c + nbuf, s)
        return ()

    lax.fori_loop(0, ncycles, ring_body, ())
    for s in range(min(nbuf, nchunks)):
        wait_write(s)


def kernel(hidden_states, base_perm, shifts):
    B, L, D = hidden_states.shape
    assert (B * L) % NW == 0 and L & (L - 1) == 0
    rows_per_w = (B * L) // NW
    CH = 32
    NBUF = 3

    hs2 = hidden_states.reshape(B * L, D)
    perm = base_perm.astype(jnp.int32)
    shifts16 = jnp.zeros((LANES,), jnp.int32).at[:B].set(shifts.astype(jnp.int32))

    mesh = plsc.VectorSubcoreMesh(core_axis_name="c", subcore_axis_name="s")
    run = pl.kernel(
        functools.partial(_body, B, L, D, rows_per_w, CH),
        out_type=jax.ShapeDtypeStruct((B * L, D), jnp.float32),
        mesh=mesh,
        scratch_types=[
            pltpu.VMEM((L,), jnp.int32),
            pltpu.VMEM((LANES,), jnp.int32),
            pltpu.VMEM((rows_per_w,), jnp.int32),
            [pltpu.VMEM((CH, D), jnp.float32) for _ in range(NBUF)],
            pltpu.SemaphoreType.DMA,
            [[pltpu.SemaphoreType.DMA for _ in range(NBUF)] for _ in range(2)],
        ],
        compiler_params=pltpu.CompilerParams(needs_layout_passes=False),
    )
    out = run(hs2, perm, shifts16)
    return out.reshape(B, L, D)


# speedup vs baseline: 1.8293x; 1.0098x over previous
import functools

import jax
import jax.numpy as jnp
from jax import lax
from jax.experimental import pallas as pl
from jax.experimental.pallas import tpu as pltpu
from jax.experimental.pallas import tpu_sc as plsc

NC, NS, LANES = 2, 16, 16
NW = NC * NS


def _body(B, L, D, rows_per_w, CH, hs_hbm, perm_hbm, shifts_hbm, out_hbm,
          perm_v, shifts_v, bufs, idx_sem, gsems, wsems):
    wid = lax.axis_index("s") * NC + lax.axis_index("c")
    base_row = wid * rows_per_w
    b = base_row // L
    r0 = base_row % L

    pltpu.async_copy(perm_hbm, perm_v, idx_sem).wait()
    pltpu.async_copy(shifts_hbm, shifts_v, idx_sem).wait()
    shift = plsc.load_gather(shifts_v, [jnp.full((LANES,), b, jnp.int32)])
    row_base = b * L

    nchunks = rows_per_w // CH
    nbuf = len(bufs)
    lanes = lax.iota(jnp.int32, LANES)

    def fire_gather(c, slot):
        pos = (lanes + shift + (r0 + c * CH)) & (L - 1)
        gidx = plsc.load_gather(perm_v, [pos]) + row_base
        pltpu.async_copy(hs_hbm.at[gidx], bufs[slot], gsems[slot])

    def wait_gather(slot):
        pltpu.make_async_copy(hs_hbm.at[pl.ds(0, CH)], bufs[slot],
                              gsems[slot]).wait()

    def fire_write(c, slot):
        pltpu.async_copy(
            bufs[slot], out_hbm.at[pl.ds(base_row + c * CH, CH)], wsems[slot])

    def wait_write(slot):
        pltpu.make_async_copy(bufs[slot], out_hbm.at[pl.ds(base_row, CH)],
                              wsems[slot]).wait()

    for s in range(min(nbuf, nchunks)):
        fire_gather(s, s)

    ncycles = -(-nchunks // nbuf)

    def ring_body(i, _):
        for s in range(nbuf):
            c = i * nbuf + s

            @pl.when(c < nchunks)
            def _():
                wait_gather(s)
                fire_write(c, s)

            @pl.when(c + nbuf < nchunks)
            def _():
                wait_write(s)
                fire_gather(c + nbuf, s)
        return ()

    lax.fori_loop(0, ncycles, ring_body, ())
    for s in range(min(nbuf, nchunks)):
        wait_write(s)


def kernel(hidden_states, base_perm, shifts):
    B, L, D = hidden_states.shape
    assert (B * L) % NW == 0 and L & (L - 1) == 0
    rows_per_w = (B * L) // NW
    CH = LANES
    NBUF = 4

    hs2 = hidden_states.reshape(B * L, D)
    perm = base_perm.astype(jnp.int32)
    shifts16 = jnp.zeros((LANES,), jnp.int32).at[:B].set(shifts.astype(jnp.int32))

    mesh = plsc.VectorSubcoreMesh(core_axis_name="c", subcore_axis_name="s")
    run = pl.kernel(
        functools.partial(_body, B, L, D, rows_per_w, CH),
        out_type=jax.ShapeDtypeStruct((B * L, D), jnp.float32),
        mesh=mesh,
        scratch_types=[
            pltpu.VMEM((L,), jnp.int32),
            pltpu.VMEM((LANES,), jnp.int32),
            [pltpu.VMEM((CH, D), jnp.float32) for _ in range(NBUF)],
            pltpu.SemaphoreType.DMA,
            [pltpu.SemaphoreType.DMA for _ in range(NBUF)],
            [pltpu.SemaphoreType.DMA for _ in range(NBUF)],
        ],
        compiler_params=pltpu.CompilerParams(needs_layout_passes=False),
    )
    out = run(hs2, perm, shifts16)
    return out.reshape(B, L, D)


# speedup vs baseline: 1.8351x; 1.0031x over previous
import functools

import jax
import jax.numpy as jnp
from jax import lax
from jax.experimental import pallas as pl
from jax.experimental.pallas import tpu as pltpu
from jax.experimental.pallas import tpu_sc as plsc

NC, NS, LANES = 2, 16, 16
NW = NC * NS


def _body(B, L, D, rows_per_w, CH, hs_hbm, perm_hbm, shifts_hbm, out_hbm,
          perm_v, shifts_v, bufs, idx_sem, gsems, wsems):
    wid = lax.axis_index("s") * NC + lax.axis_index("c")
    base_row = wid * rows_per_w
    b = base_row // L
    r0 = base_row % L

    pltpu.async_copy(perm_hbm, perm_v, idx_sem)
    pltpu.async_copy(shifts_hbm, shifts_v, idx_sem)
    pltpu.make_async_copy(perm_hbm, perm_v, idx_sem).wait()
    pltpu.make_async_copy(shifts_hbm, shifts_v, idx_sem).wait()
    shift = plsc.load_gather(shifts_v, [jnp.full((LANES,), b, jnp.int32)])
    row_base = b * L

    nchunks = rows_per_w // CH
    nbuf = len(bufs)
    lanes = lax.iota(jnp.int32, LANES)

    def fire_gather(c, slot):
        pos = (lanes + shift + (r0 + c * CH)) & (L - 1)
        gidx = plsc.load_gather(perm_v, [pos]) + row_base
        pltpu.async_copy(hs_hbm.at[gidx], bufs[slot], gsems[slot])

    def wait_gather(slot):
        pltpu.make_async_copy(hs_hbm.at[pl.ds(0, CH)], bufs[slot],
                              gsems[slot]).wait()

    def fire_write(c, slot):
        pltpu.async_copy(
            bufs[slot], out_hbm.at[pl.ds(base_row + c * CH, CH)], wsems[slot])

    def wait_write(slot):
        pltpu.make_async_copy(bufs[slot], out_hbm.at[pl.ds(base_row, CH)],
                              wsems[slot]).wait()

    for s in range(min(nbuf, nchunks)):
        fire_gather(s, s)

    ncycles = -(-nchunks // nbuf)

    def ring_body(i, _):
        for s in range(nbuf):
            c = i * nbuf + s

            @pl.when(c < nchunks)
            def _():
                wait_gather(s)
                fire_write(c, s)

            @pl.when(c + nbuf < nchunks)
            def _():
                wait_write(s)
                fire_gather(c + nbuf, s)
        return ()

    lax.fori_loop(0, ncycles, ring_body, ())
    for s in range(min(nbuf, nchunks)):
        wait_write(s)


def kernel(hidden_states, base_perm, shifts):
    B, L, D = hidden_states.shape
    assert (B * L) % NW == 0 and L & (L - 1) == 0
    rows_per_w = (B * L) // NW
    CH = LANES
    NBUF = 6

    hs2 = hidden_states.reshape(B * L, D)
    perm = base_perm.astype(jnp.int32)
    shifts_i = shifts.astype(jnp.int32)

    mesh = plsc.VectorSubcoreMesh(core_axis_name="c", subcore_axis_name="s")
    run = pl.kernel(
        functools.partial(_body, B, L, D, rows_per_w, CH),
        out_type=jax.ShapeDtypeStruct((B * L, D), jnp.float32),
        mesh=mesh,
        scratch_types=[
            pltpu.VMEM((L,), jnp.int32),
            pltpu.VMEM((B,), jnp.int32),
            [pltpu.VMEM((CH, D), jnp.float32) for _ in range(NBUF)],
            pltpu.SemaphoreType.DMA,
            [pltpu.SemaphoreType.DMA for _ in range(NBUF)],
            [pltpu.SemaphoreType.DMA for _ in range(NBUF)],
        ],
        compiler_params=pltpu.CompilerParams(needs_layout_passes=False),
    )
    out = run(hs2, perm, shifts_i)
    return out.reshape(B, L, D)
